# baseline (speedup 1.0000x reference)
import functools

import jax
import jax.numpy as jnp
from jax import lax
from jax.experimental import pallas as pl
from jax.experimental.pallas import tpu as pltpu
from jax.experimental.pallas import tpu_sc as plsc

N_NODE = 100000
EMB_DIM = 128
BATCH = 16384

NC = 2
NS = 16
NW = NC * NS
ROWS_PER_W = BATCH // NW
CH = 128
NCH = ROWS_PER_W // CH
NBUF = 3


def _disc_body(node_id, nbr_id, emb, bias_vec,
               score_o, ne_o, nn_o, bias_o,
               idx_a, idx_b, bias_v, score_v, tp, a_all, b_all,
               *sems):
    gaa = sems[0:3]
    gab = sems[3:6]
    oa = sems[6:9]
    ob = sems[9:12]
    sbias = sems[12]

    wid = lax.axis_index("s") * NC + lax.axis_index("c")
    iota16 = lax.iota(jnp.int32, 16) * 16
    base = wid * ROWS_PER_W

    def a_buf(k):
        return a_all.at[pl.ds(k * CH, CH), :]

    def b_buf(k):
        return b_all.at[pl.ds(k * CH, CH), :]

    def start_gather(i, k):
        pltpu.async_copy(emb.at[idx_a.at[pl.ds(i * CH, CH)]], a_buf(k), gaa[k])
        pltpu.async_copy(emb.at[idx_b.at[pl.ds(i * CH, CH)]], b_buf(k), gab[k])

    def wait_gather(i, k):
        pltpu.make_async_copy(emb.at[idx_a.at[pl.ds(i * CH, CH)]], a_buf(k), gaa[k]).wait()
        pltpu.make_async_copy(emb.at[idx_b.at[pl.ds(i * CH, CH)]], b_buf(k), gab[k]).wait()

    def start_out(i, k):
        dst = pl.multiple_of(base + i * CH, CH)
        pltpu.async_copy(a_buf(k), ne_o.at[pl.ds(dst, CH)], oa[k])
        pltpu.async_copy(b_buf(k), nn_o.at[pl.ds(dst, CH)], ob[k])

    def wait_out(i, k):
        dst = pl.multiple_of(base + i * CH, CH)
        pltpu.make_async_copy(a_buf(k), ne_o.at[pl.ds(dst, CH)], oa[k]).wait()
        pltpu.make_async_copy(b_buf(k), nn_o.at[pl.ds(dst, CH)], ob[k]).wait()

    pltpu.sync_copy(node_id.at[pl.ds(base, ROWS_PER_W)], idx_a)
    pltpu.sync_copy(nbr_id.at[pl.ds(base, ROWS_PER_W)], idx_b)

    cp_bias = pltpu.async_copy(bias_vec.at[idx_b], bias_v, sbias)
    start_gather(0, 0)
    start_gather(1, 1)
    cp_bias.wait()

    def chunk_body(i, carry):
        buf = lax.rem(i, NBUF)
        for k in range(NBUF):
            @pl.when(buf == k)
            def _(k=k):
                wait_gather(i, k)

                @pl.when(i == 1)
                def _():
                    wait_out(0, (k + 2) % NBUF)

                @pl.when(i < NCH - 2)
                def _():
                    start_gather(i + 2, (k + 2) % NBUF)

        rowbase = buf * CH
        cbase = i * CH

        def group_body(g, carry2):
            gbase = g * 16

            def row_body(r, carry3):
                row = rowbase + gbase + r
                acc = a_all[row, pl.ds(0, 16)] * b_all[row, pl.ds(0, 16)]
                for c in range(1, 8):
                    acc = acc + (a_all[row, pl.ds(c * 16, 16)]
                                 * b_all[row, pl.ds(c * 16, 16)])
                tp[pl.ds(r * 16, 16)] = acc
                return carry3

            lax.fori_loop(0, 16, row_body, 0)

            def tsum_body(j, s):
                return s + plsc.load_gather(tp, [iota16 + j])

            s = lax.fori_loop(1, 16, tsum_body, plsc.load_gather(tp, [iota16]))
            s = s + bias_v[pl.ds(cbase + gbase, 16)]
            s = jnp.minimum(jnp.maximum(s, -10.0), 10.0)
            score_v[pl.ds(cbase + gbase, 16)] = s
            return carry2

        lax.fori_loop(0, CH // 16, group_body, 0)

        for k in range(NBUF):
            @pl.when(buf == k)
            def _(k=k):
                start_out(i, k)
        return carry

    lax.fori_loop(0, NCH, chunk_body, 0)

    pltpu.sync_copy(bias_v, bias_o.at[pl.ds(base, ROWS_PER_W)])
    pltpu.sync_copy(score_v, score_o.at[pl.ds(base, ROWS_PER_W)])
    wait_out(1, 1)
    wait_out(2, 2)
    wait_out(3, 0)


_disc = functools.partial(
    pl.kernel,
    out_type=(
        jax.ShapeDtypeStruct((BATCH,), jnp.float32),
        jax.ShapeDtypeStruct((BATCH, EMB_DIM), jnp.float32),
        jax.ShapeDtypeStruct((BATCH, EMB_DIM), jnp.float32),
        jax.ShapeDtypeStruct((BATCH,), jnp.float32),
    ),
    mesh=plsc.VectorSubcoreMesh(core_axis_name="c", subcore_axis_name="s",
                                num_cores=NC, num_subcores=NS),
    compiler_params=pltpu.CompilerParams(needs_layout_passes=False),
    scratch_types=(
        [
            pltpu.VMEM((ROWS_PER_W,), jnp.int32),
            pltpu.VMEM((ROWS_PER_W,), jnp.int32),
            pltpu.VMEM((ROWS_PER_W,), jnp.float32),
            pltpu.VMEM((ROWS_PER_W,), jnp.float32),
            pltpu.VMEM((256,), jnp.float32),
            pltpu.VMEM((NBUF * CH, EMB_DIM), jnp.float32),
            pltpu.VMEM((NBUF * CH, EMB_DIM), jnp.float32),
        ]
        + [pltpu.SemaphoreType.DMA] * 13
    ),
)(_disc_body)


@jax.jit
def kernel(node_id, node_neighbor_id, embedding_matrix, bias_vector):
    score, ne, nn, bias = _disc(node_id, node_neighbor_id,
                                embedding_matrix, bias_vector)
    return (score, ne, nn, bias)
